# Initial kernel scaffold; baseline (speedup 1.0000x reference)
import jax
import jax.numpy as jnp
from jax.experimental import pallas as pl


def kernel(tokens, W_pos):
    raise NotImplementedError("write your pallas kernel here")


# speedup vs baseline: 1.1956x; 1.1956x over previous
import jax
import jax.numpy as jnp
from jax.experimental import pallas as pl


def kernel(tokens, W_pos):
    B, P = tokens.shape
    D = W_pos.shape[1]
    BLK = 512

    def body(w_ref, o_ref):
        o_ref[0] = w_ref[...]

    out = pl.pallas_call(
        body,
        grid=(P // BLK, B),
        in_specs=[pl.BlockSpec((BLK, D), lambda p, b: (p, 0))],
        out_specs=pl.BlockSpec((1, BLK, D), lambda p, b: (b, p, 0)),
        out_shape=jax.ShapeDtypeStruct((B, P, D), W_pos.dtype),
    )(W_pos)
    return out


# speedup vs baseline: 1.3210x; 1.1048x over previous
import jax
import jax.numpy as jnp
from jax.experimental import pallas as pl


def kernel(tokens, W_pos):
    B, P = tokens.shape
    D = W_pos.shape[1]
    BLK = 1024

    def body(w_ref, o_ref):
        o_ref[0] = w_ref[...]

    out = pl.pallas_call(
        body,
        grid=(P // BLK, B),
        in_specs=[pl.BlockSpec((BLK, D), lambda p, b: (p, 0))],
        out_specs=pl.BlockSpec((1, BLK, D), lambda p, b: (b, p, 0)),
        out_shape=jax.ShapeDtypeStruct((B, P, D), W_pos.dtype),
    )(W_pos)
    return out


# speedup vs baseline: 1.4093x; 1.0668x over previous
import jax
import jax.numpy as jnp
from jax.experimental import pallas as pl
from jax.experimental.pallas import tpu as pltpu


def kernel(tokens, W_pos):
    B, P = tokens.shape
    D = W_pos.shape[1]
    CHUNK = 512
    NC = P // CHUNK
    NB = 2

    def body(w_hbm, o_hbm, buf, in_sem, out_sem):
        def in_copy(i):
            return pltpu.make_async_copy(
                w_hbm.at[pl.ds(i * CHUNK, CHUNK), :], buf.at[i % NB],
                in_sem.at[i % NB])

        def out_copy(i, b):
            return pltpu.make_async_copy(
                buf.at[i % NB], o_hbm.at[b, pl.ds(i * CHUNK, CHUNK), :],
                out_sem.at[i % NB, b])

        in_copy(0).start()
        for i in range(NC):
            if i + 1 < NC:
                if i >= 1:
                    for b in range(B):
                        out_copy(i - 1, b).wait()
                in_copy(i + 1).start()
            in_copy(i).wait()
            for b in range(B):
                out_copy(i, b).start()
        for i in (NC - 2, NC - 1):
            if i >= 0:
                for b in range(B):
                    out_copy(i, b).wait()

    out = pl.pallas_call(
        body,
        in_specs=[pl.BlockSpec(memory_space=pl.ANY)],
        out_specs=pl.BlockSpec(memory_space=pl.ANY),
        out_shape=jax.ShapeDtypeStruct((B, P, D), W_pos.dtype),
        scratch_shapes=[
            pltpu.VMEM((NB, CHUNK, D), W_pos.dtype),
            pltpu.SemaphoreType.DMA((NB,)),
            pltpu.SemaphoreType.DMA((NB, B)),
        ],
    )(W_pos)
    return out


# speedup vs baseline: 1.5128x; 1.0735x over previous
import jax
import jax.numpy as jnp
from jax.experimental import pallas as pl
from jax.experimental.pallas import tpu as pltpu


def kernel(tokens, W_pos):
    B, P = tokens.shape
    D = W_pos.shape[1]
    CHUNK = 512
    NC = P // CHUNK

    def body(w_hbm, o_hbm, buf, in_sem, out_sem):
        def in_copy(i):
            return pltpu.make_async_copy(
                w_hbm.at[pl.ds(i * CHUNK, CHUNK), :],
                buf.at[pl.ds(i * CHUNK, CHUNK), :],
                in_sem.at[i])

        def out_copy(i, b):
            return pltpu.make_async_copy(
                buf.at[pl.ds(i * CHUNK, CHUNK), :],
                o_hbm.at[b, pl.ds(i * CHUNK, CHUNK), :],
                out_sem.at[i, b])

        for i in range(NC):
            in_copy(i).start()
        for i in range(NC):
            in_copy(i).wait()
            for b in range(B):
                out_copy(i, b).start()
        for i in range(NC):
            for b in range(B):
                out_copy(i, b).wait()

    out = pl.pallas_call(
        body,
        in_specs=[pl.BlockSpec(memory_space=pl.ANY)],
        out_specs=pl.BlockSpec(memory_space=pl.ANY),
        out_shape=jax.ShapeDtypeStruct((B, P, D), W_pos.dtype),
        scratch_shapes=[
            pltpu.VMEM((P, D), W_pos.dtype),
            pltpu.SemaphoreType.DMA((NC,)),
            pltpu.SemaphoreType.DMA((NC, B)),
        ],
    )(W_pos)
    return out


# speedup vs baseline: 1.5327x; 1.0131x over previous
import jax
import jax.numpy as jnp
from jax.experimental import pallas as pl
from jax.experimental.pallas import tpu as pltpu


def kernel(tokens, W_pos):
    B, P = tokens.shape
    D = W_pos.shape[1]
    CHUNK = 1024
    NC = P // CHUNK

    def body(w_hbm, o_hbm, buf, in_sem, out_sem):
        def in_copy(i):
            return pltpu.make_async_copy(
                w_hbm.at[pl.ds(i * CHUNK, CHUNK), :],
                buf.at[pl.ds(i * CHUNK, CHUNK), :],
                in_sem.at[i])

        def out_copy(i, b):
            return pltpu.make_async_copy(
                buf.at[pl.ds(i * CHUNK, CHUNK), :],
                o_hbm.at[b, pl.ds(i * CHUNK, CHUNK), :],
                out_sem.at[i, b])

        for i in range(NC):
            in_copy(i).start()
        for i in range(NC):
            in_copy(i).wait()
            for b in range(B):
                out_copy(i, b).start()
        for i in range(NC):
            for b in range(B):
                out_copy(i, b).wait()

    out = pl.pallas_call(
        body,
        in_specs=[pl.BlockSpec(memory_space=pl.ANY)],
        out_specs=pl.BlockSpec(memory_space=pl.ANY),
        out_shape=jax.ShapeDtypeStruct((B, P, D), W_pos.dtype),
        scratch_shapes=[
            pltpu.VMEM((P, D), W_pos.dtype),
            pltpu.SemaphoreType.DMA((NC,)),
            pltpu.SemaphoreType.DMA((NC, B)),
        ],
    )(W_pos)
    return out
